# baseline (speedup 1.0000x reference)
import functools

import jax
import jax.numpy as jnp
from jax import lax
from jax.experimental import pallas as pl
from jax.experimental.pallas import tpu as pltpu
from jax.experimental.pallas import tpu_sc as plsc

VOCAB = 1000000
EMBED = 64
HALF = EMBED // 2
LANES = 128
BATCH = 4096
WINDOW = 200

TOTAL = 2 * BATCH * WINDOW
NUM_CORES = 2
NUM_SUBCORES = 16
NW = NUM_CORES * NUM_SUBCORES
PER_W = TOTAL // NW
CHUNK = 160
NBUF = 4
GROUP = NBUF * CHUNK
NGROUP = PER_W // GROUP

_mesh = plsc.VectorSubcoreMesh(
    core_axis_name="c", subcore_axis_name="s",
    num_cores=NUM_CORES, num_subcores=NUM_SUBCORES,
)


@functools.partial(
    pl.kernel,
    out_type=jax.ShapeDtypeStruct((4 * TOTAL, HALF), jnp.float32),
    mesh=_mesh,
    scratch_types=[
        pltpu.VMEM((NBUF * 4 * CHUNK,), jnp.int32),
        pltpu.VMEM((NBUF, 4 * CHUNK, HALF), jnp.float32),
        pltpu.SemaphoreType.DMA,
        pltpu.SemaphoreType.DMA,
        pltpu.SemaphoreType.DMA,
    ],
    compiler_params=pltpu.CompilerParams(use_tc_tiling_on_sc=False),
)
def _gather_all(table_hbm, idx4_hbm, out_hbm, idx_v, rows_v, sem_i, sem_g,
                sem_s):
    wid = lax.axis_index("s") * NUM_CORES + lax.axis_index("c")
    base = wid * PER_W

    def group(g, carry):
        goff = base + g * GROUP
        lds = [
            pltpu.async_copy(
                idx4_hbm.at[pl.ds(4 * (goff + b * CHUNK), 4 * CHUNK)],
                idx_v.at[pl.ds(b * 4 * CHUNK, 4 * CHUNK)], sem_i)
            for b in range(NBUF)
        ]
        gat = []
        for b in range(NBUF):
            lds[b].wait()
            gat.append(pltpu.async_copy(
                table_hbm.at[idx_v.at[pl.ds(b * 4 * CHUNK, 4 * CHUNK)]],
                rows_v.at[b], sem_g))
        sto = []
        for b in range(NBUF):
            gat[b].wait()
            sto.append(pltpu.async_copy(
                rows_v.at[b],
                out_hbm.at[pl.ds(4 * (goff + b * CHUNK), 4 * CHUNK)],
                sem_s))
        for d in sto:
            d.wait()
        return carry

    lax.fori_loop(0, NGROUP, group, 0)


def kernel(inputs, labels, E):
    idx = jnp.concatenate(
        [inputs.reshape(-1), labels.reshape(-1)]).astype(jnp.int32)
    idx4 = (2 * idx[:, None]
            + jnp.array([0, 1, 0, 1], dtype=jnp.int32)).reshape(-1)
    table = E.reshape(2 * VOCAB, HALF)
    out = _gather_all(table, idx4)
    return out.reshape(TOTAL, LANES)[:, :EMBED].reshape(
        2, BATCH, WINDOW, EMBED)


# speedup vs baseline: 1.9506x; 1.9506x over previous
import functools

import jax
import jax.numpy as jnp
from jax import lax
from jax.experimental import pallas as pl
from jax.experimental.pallas import tpu as pltpu
from jax.experimental.pallas import tpu_sc as plsc

VOCAB = 1000000
EMBED = 64
HALF = EMBED // 2
LANES = 128
BATCH = 4096
WINDOW = 200

TOTAL = 2 * BATCH * WINDOW
NUM_CORES = 2
NUM_SUBCORES = 16
NW = NUM_CORES * NUM_SUBCORES
PER_W = TOTAL // NW
CHUNK = 256
NBUF = 4
GROUP = NBUF * CHUNK
NGROUP = PER_W // GROUP
L = 16

_mesh = plsc.VectorSubcoreMesh(
    core_axis_name="c", subcore_axis_name="s",
    num_cores=NUM_CORES, num_subcores=NUM_SUBCORES,
)


@functools.partial(
    pl.kernel,
    out_type=jax.ShapeDtypeStruct((4 * TOTAL, HALF), jnp.float32),
    mesh=_mesh,
    scratch_types=[
        pltpu.VMEM((NBUF * CHUNK,), jnp.int32),
        pltpu.VMEM((NBUF * 2 * CHUNK,), jnp.int32),
        pltpu.VMEM((2 * CHUNK,), jnp.int32),
        pltpu.VMEM((NBUF, 2 * CHUNK, HALF), jnp.float32),
        pltpu.SemaphoreType.DMA,
        pltpu.SemaphoreType.DMA,
        pltpu.SemaphoreType.DMA,
    ],
    compiler_params=pltpu.CompilerParams(
        use_tc_tiling_on_sc=False, needs_layout_passes=False),
)
def _gather_all(table_hbm, idx_hbm, out_hbm, idx_v, idx2_v, patt_v, rows_v,
                sem_i, sem_g, sem_s):
    wid = lax.axis_index("s") * NUM_CORES + lax.axis_index("c")
    base = wid * PER_W
    iota = lax.iota(jnp.int32, L)

    def patt(k, carry):
        j = k * L + iota
        plsc.store_scatter(patt_v, [2 * k * L + 2 * iota], 4 * j)
        plsc.store_scatter(patt_v, [2 * k * L + 2 * iota + 1], 4 * j + 1)
        return carry

    lax.fori_loop(0, CHUNK // L, patt, 0)

    def group(g, carry):
        goff = base + g * GROUP
        lds = [
            pltpu.async_copy(
                idx_hbm.at[pl.ds(goff + b * CHUNK, CHUNK)],
                idx_v.at[pl.ds(b * CHUNK, CHUNK)], sem_i)
            for b in range(NBUF)
        ]
        gat = []
        for b in range(NBUF):
            lds[b].wait()

            def expand(k, carry, b=b):
                src = idx_v[pl.ds(b * CHUNK + k * L, L)]
                e = 2 * src
                p = 2 * (b * CHUNK + k * L) + 2 * iota
                plsc.store_scatter(idx2_v, [p], e)
                plsc.store_scatter(idx2_v, [p + 1], e + 1)
                return carry

            lax.fori_loop(0, CHUNK // L, expand, 0)
            gat.append(pltpu.async_copy(
                table_hbm.at[idx2_v.at[pl.ds(b * 2 * CHUNK, 2 * CHUNK)]],
                rows_v.at[b], sem_g))
        sto = []
        for b in range(NBUF):
            gat[b].wait()
            sto.append(pltpu.async_copy(
                rows_v.at[b],
                out_hbm.at[pl.ds(4 * (goff + b * CHUNK), 4 * CHUNK)]
                .at[patt_v],
                sem_s))
        for d in sto:
            d.wait()
        return carry

    lax.fori_loop(0, NGROUP, group, 0)


def kernel(inputs, labels, E):
    idx = jnp.concatenate(
        [inputs.reshape(-1), labels.reshape(-1)]).astype(jnp.int32)
    table = E.reshape(2 * VOCAB, HALF)
    out = _gather_all(table, idx)
    return out.reshape(TOTAL, LANES)[:, :EMBED].reshape(
        2, BATCH, WINDOW, EMBED)
